# Initial kernel scaffold; baseline (speedup 1.0000x reference)
import jax
import jax.numpy as jnp
from jax.experimental import pallas as pl


def kernel(x, adj, M, W1, a_self1, a_neighs1, W2, a_self2, a_neighs2, cluster_layer):
    raise NotImplementedError("write your pallas kernel here")


# speedup vs baseline: 31.8332x; 31.8332x over previous
import functools

import jax
import jax.numpy as jnp
from jax.experimental import pallas as pl

ALPHA = 0.2
NEG = -9e15
NBINS = 100


def _proj_body(x_ref, w_ref, asf_ref, anb_ref, h_ref, ss_ref, sn_ref):
    h = jnp.dot(x_ref[...], w_ref[...], preferred_element_type=jnp.float32)
    h_ref[...] = h
    ss_ref[...] = jnp.dot(h, asf_ref[...], preferred_element_type=jnp.float32)
    sn_ref[...] = jnp.dot(h, anb_ref[...], preferred_element_type=jnp.float32)


def _proj(x, w, a_self, a_neighs, bn):
    n, f = x.shape
    hd = w.shape[1]
    return pl.pallas_call(
        _proj_body,
        grid=(n // bn,),
        in_specs=[
            pl.BlockSpec((bn, f), lambda i: (i, 0)),
            pl.BlockSpec((f, hd), lambda i: (0, 0)),
            pl.BlockSpec((hd, 1), lambda i: (0, 0)),
            pl.BlockSpec((hd, 1), lambda i: (0, 0)),
        ],
        out_specs=[
            pl.BlockSpec((bn, hd), lambda i: (i, 0)),
            pl.BlockSpec((bn, 1), lambda i: (i, 0)),
            pl.BlockSpec((bn, 1), lambda i: (i, 0)),
        ],
        out_shape=[
            jax.ShapeDtypeStruct((n, hd), jnp.float32),
            jax.ShapeDtypeStruct((n, 1), jnp.float32),
            jax.ShapeDtypeStruct((n, 1), jnp.float32),
        ],
    )(x, w, a_self, a_neighs)


def _att_body(ss_ref, sn_ref, m_ref, adj_ref, h_ref, out_ref, *, normalize):
    e = ss_ref[...] + sn_ref[...]
    e = e * m_ref[...]
    e = jnp.where(e > 0, e, ALPHA * e)
    e = jnp.where(adj_ref[...] > 0, e, NEG)
    mx = jnp.max(e, axis=1, keepdims=True)
    p = jnp.exp(e - mx)
    att = p / jnp.sum(p, axis=1, keepdims=True)
    h2 = jnp.dot(att, h_ref[...], preferred_element_type=jnp.float32)
    a = jnp.where(h2 > 0, h2, jnp.exp(jnp.minimum(h2, 0.0)) - 1.0)
    if normalize:
        nrm = jnp.sqrt(jnp.sum(a * a, axis=1, keepdims=True))
        a = a / jnp.maximum(nrm, 1e-12)
    out_ref[...] = a


def _att(ss, sn_row, m, adj, h, bn, normalize):
    n = m.shape[0]
    hd = h.shape[1]
    return pl.pallas_call(
        functools.partial(_att_body, normalize=normalize),
        grid=(n // bn,),
        in_specs=[
            pl.BlockSpec((bn, 1), lambda i: (i, 0)),
            pl.BlockSpec((1, n), lambda i: (0, 0)),
            pl.BlockSpec((bn, n), lambda i: (i, 0)),
            pl.BlockSpec((bn, n), lambda i: (i, 0)),
            pl.BlockSpec((n, hd), lambda i: (0, 0)),
        ],
        out_specs=pl.BlockSpec((bn, hd), lambda i: (i, 0)),
        out_shape=jax.ShapeDtypeStruct((n, hd), jnp.float32),
    )(ss, sn_row, m, adj, h)


def _apred_body(z_ref, zf_ref, out_ref):
    s = jax.lax.dot_general(
        z_ref[...], zf_ref[...], (((1,), (1,)), ((), ())),
        preferred_element_type=jnp.float32)
    out_ref[...] = 1.0 / (1.0 + jnp.exp(-s))


def _apred(z, bn):
    n, e = z.shape
    return pl.pallas_call(
        _apred_body,
        grid=(n // bn,),
        in_specs=[
            pl.BlockSpec((bn, e), lambda i: (i, 0)),
            pl.BlockSpec((n, e), lambda i: (0, 0)),
        ],
        out_specs=pl.BlockSpec((bn, n), lambda i: (i, 0)),
        out_shape=jax.ShapeDtypeStruct((n, n), jnp.float32),
    )(z, z)


def _hist_body(z_ref, clt_ref, bz_ref, pt_ref):
    z = z_ref[...]
    clt = clt_ref[...]
    n, f = z.shape
    c = clt.shape[1]
    mn = jnp.minimum(jnp.min(z, axis=0, keepdims=True),
                     jnp.reshape(jnp.min(clt, axis=1, keepdims=True), (1, f)))
    mx = jnp.maximum(jnp.max(z, axis=0, keepdims=True),
                     jnp.reshape(jnp.max(clt, axis=1, keepdims=True), (1, f)))
    step = (mx - mn) / NBINS
    bz = jnp.clip(jnp.floor((z - mn) / step), 0, NBINS - 1).astype(jnp.int32)
    mnt = jnp.reshape(mn, (f, 1))
    stept = jnp.reshape(step, (f, 1))
    bct = jnp.clip(jnp.floor((clt - mnt) / stept), 0, NBINS - 1).astype(jnp.int32)
    bz_ref[...] = bz

    rows = []
    iota_n = jax.lax.broadcasted_iota(jnp.int32, (n, NBINS), 1)
    iota_c = jax.lax.broadcasted_iota(jnp.int32, (c, NBINS), 1)
    for j in range(f):
        ohz = (bz[:, j:j + 1] == iota_n).astype(jnp.float32)
        ohc = (jnp.reshape(bct[j:j + 1, :], (c, 1)) == iota_c).astype(jnp.float32)
        rows.append(jnp.sum(ohz, axis=0, keepdims=True)
                    + jnp.sum(ohc, axis=0, keepdims=True))
    counts = jnp.concatenate(rows, axis=0)

    s4 = jax.lax.broadcasted_iota(jnp.int32, (f, NBINS, c, NBINS), 1)
    k4 = jax.lax.broadcasted_iota(jnp.int32, (f, NBINS, c, NBINS), 3)
    bc4 = jnp.reshape(bct, (f, 1, c, 1))
    hi = jnp.maximum(s4, bc4)
    lo = jnp.minimum(s4, bc4)
    w = ((k4 <= hi).astype(jnp.float32)
         - (lo > 0).astype(jnp.float32) * (k4 <= lo - 1).astype(jnp.float32))
    mass = jnp.sum(jnp.reshape(counts, (f, 1, 1, NBINS)) * w, axis=3)
    p = mass / float(n + c)
    pt_ref[...] = jnp.reshape(p * p, (f * NBINS, c))


def _hist(z, clt):
    n, f = z.shape
    c = clt.shape[1]
    return pl.pallas_call(
        _hist_body,
        out_shape=[
            jax.ShapeDtypeStruct((n, f), jnp.int32),
            jax.ShapeDtypeStruct((f * NBINS, c), jnp.float32),
        ],
    )(z, clt)


def _q_body(bz_ref, pt_ref, q_ref):
    bz = bz_ref[...]
    bn, f = bz.shape
    oh = (bz[:, :, None]
          == jax.lax.broadcasted_iota(jnp.int32, (bn, f, NBINS), 2))
    oh = jnp.reshape(oh.astype(jnp.float32), (bn, f * NBINS))
    dmass = jnp.dot(oh, pt_ref[...], preferred_element_type=jnp.float32)
    dm = jnp.sqrt(dmass)
    qq = 1.0 / (1.0 + dm)
    q_ref[...] = qq / jnp.sum(qq, axis=1, keepdims=True)


def _q(bz, pt, bn):
    n, f = bz.shape
    c = pt.shape[1]
    return pl.pallas_call(
        _q_body,
        grid=(n // bn,),
        in_specs=[
            pl.BlockSpec((bn, f), lambda i: (i, 0)),
            pl.BlockSpec((f * NBINS, c), lambda i: (0, 0)),
        ],
        out_specs=pl.BlockSpec((bn, c), lambda i: (i, 0)),
        out_shape=jax.ShapeDtypeStruct((n, c), jnp.float32),
    )(bz, pt)


def kernel(x, adj, M, W1, a_self1, a_neighs1, W2, a_self2, a_neighs2,
           cluster_layer):
    h1, ss1, sn1 = _proj(x, W1, a_self1, a_neighs1, 512)
    h1 = _att(ss1, jnp.reshape(sn1, (1, -1)), M, adj, h1, 128, False)
    h2, ss2, sn2 = _proj(h1, W2, a_self2, a_neighs2, 512)
    z = _att(ss2, jnp.reshape(sn2, (1, -1)), M, adj, h2, 128, True)
    a_pred = _apred(z, 256)
    bz, pt = _hist(z, jnp.transpose(cluster_layer))
    q = _q(bz, pt, 512)
    return (a_pred, z, q)


# speedup vs baseline: 35.0499x; 1.1010x over previous
import functools

import jax
import jax.numpy as jnp
from jax.experimental import pallas as pl

ALPHA = 0.2
NEG = -9e15
NBINS = 100


def _proj_body(x_ref, w_ref, asf_ref, anb_ref, h_ref, ss_ref, sn_ref):
    h = jnp.dot(x_ref[...], w_ref[...], preferred_element_type=jnp.float32)
    h_ref[...] = h
    ss_ref[...] = jnp.dot(h, asf_ref[...], preferred_element_type=jnp.float32)
    sn_ref[...] = jnp.dot(h, anb_ref[...], preferred_element_type=jnp.float32)


def _proj(x, w, a_self, a_neighs, bn):
    n, f = x.shape
    hd = w.shape[1]
    return pl.pallas_call(
        _proj_body,
        grid=(n // bn,),
        in_specs=[
            pl.BlockSpec((bn, f), lambda i: (i, 0)),
            pl.BlockSpec((f, hd), lambda i: (0, 0)),
            pl.BlockSpec((hd, 1), lambda i: (0, 0)),
            pl.BlockSpec((hd, 1), lambda i: (0, 0)),
        ],
        out_specs=[
            pl.BlockSpec((bn, hd), lambda i: (i, 0)),
            pl.BlockSpec((bn, 1), lambda i: (i, 0)),
            pl.BlockSpec((bn, 1), lambda i: (i, 0)),
        ],
        out_shape=[
            jax.ShapeDtypeStruct((n, hd), jnp.float32),
            jax.ShapeDtypeStruct((n, 1), jnp.float32),
            jax.ShapeDtypeStruct((n, 1), jnp.float32),
        ],
    )(x, w, a_self, a_neighs)


def _att_body(ss_ref, sn_ref, m_ref, adj_ref, h_ref, out_ref, *, normalize):
    e = ss_ref[...] + sn_ref[...]
    e = e * m_ref[...]
    e = jnp.where(e > 0, e, ALPHA * e)
    e = jnp.where(adj_ref[...] > 0, e, NEG)
    mx = jnp.max(e, axis=1, keepdims=True)
    p = jnp.exp(e - mx)
    l = jnp.sum(p, axis=1, keepdims=True)
    h2 = jnp.dot(p.astype(jnp.bfloat16), h_ref[...].astype(jnp.bfloat16),
                 preferred_element_type=jnp.float32) / l
    a = jnp.where(h2 > 0, h2, jnp.exp(jnp.minimum(h2, 0.0)) - 1.0)
    if normalize:
        nrm = jnp.sqrt(jnp.sum(a * a, axis=1, keepdims=True))
        a = a / jnp.maximum(nrm, 1e-12)
    out_ref[...] = a


def _att(ss, sn_row, m, adj, h, bn, normalize):
    n = m.shape[0]
    hd = h.shape[1]
    return pl.pallas_call(
        functools.partial(_att_body, normalize=normalize),
        grid=(n // bn,),
        in_specs=[
            pl.BlockSpec((bn, 1), lambda i: (i, 0)),
            pl.BlockSpec((1, n), lambda i: (0, 0)),
            pl.BlockSpec((bn, n), lambda i: (i, 0)),
            pl.BlockSpec((bn, n), lambda i: (i, 0)),
            pl.BlockSpec((n, hd), lambda i: (0, 0)),
        ],
        out_specs=pl.BlockSpec((bn, hd), lambda i: (i, 0)),
        out_shape=jax.ShapeDtypeStruct((n, hd), jnp.float32),
    )(ss, sn_row, m, adj, h)


def _apred_body(z_ref, zf_ref, out_ref):
    s = jax.lax.dot_general(
        z_ref[...].astype(jnp.bfloat16), zf_ref[...].astype(jnp.bfloat16),
        (((1,), (1,)), ((), ())), preferred_element_type=jnp.float32)
    out_ref[...] = 1.0 / (1.0 + jnp.exp(-s))


def _apred(z, bn):
    n, e = z.shape
    return pl.pallas_call(
        _apred_body,
        grid=(n // bn,),
        in_specs=[
            pl.BlockSpec((bn, e), lambda i: (i, 0)),
            pl.BlockSpec((n, e), lambda i: (0, 0)),
        ],
        out_specs=pl.BlockSpec((bn, n), lambda i: (i, 0)),
        out_shape=jax.ShapeDtypeStruct((n, n), jnp.float32),
    )(z, z)


def _hist_body(z_ref, clt_ref, bz_ref, pt_ref):
    z = z_ref[...]
    clt = clt_ref[...]
    n, f = z.shape
    c = clt.shape[1]
    mn = jnp.minimum(jnp.min(z, axis=0, keepdims=True),
                     jnp.reshape(jnp.min(clt, axis=1, keepdims=True), (1, f)))
    mx = jnp.maximum(jnp.max(z, axis=0, keepdims=True),
                     jnp.reshape(jnp.max(clt, axis=1, keepdims=True), (1, f)))
    step = (mx - mn) / NBINS
    bz = jnp.clip(jnp.floor((z - mn) / step), 0, NBINS - 1).astype(jnp.int32)
    mnt = jnp.reshape(mn, (f, 1))
    stept = jnp.reshape(step, (f, 1))
    bct = jnp.clip(jnp.floor((clt - mnt) / stept), 0, NBINS - 1).astype(jnp.int32)
    bz_ref[...] = bz

    rows = []
    iota_n = jax.lax.broadcasted_iota(jnp.int32, (n, NBINS), 1)
    iota_c = jax.lax.broadcasted_iota(jnp.int32, (c, NBINS), 1)
    for j in range(f):
        ohz = (bz[:, j:j + 1] == iota_n).astype(jnp.float32)
        ohc = (jnp.reshape(bct[j:j + 1, :], (c, 1)) == iota_c).astype(jnp.float32)
        rows.append(jnp.sum(ohz, axis=0, keepdims=True)
                    + jnp.sum(ohc, axis=0, keepdims=True))
    counts = jnp.concatenate(rows, axis=0)

    s4 = jax.lax.broadcasted_iota(jnp.int32, (f, NBINS, c, NBINS), 1)
    k4 = jax.lax.broadcasted_iota(jnp.int32, (f, NBINS, c, NBINS), 3)
    bc4 = jnp.reshape(bct, (f, 1, c, 1))
    hi = jnp.maximum(s4, bc4)
    lo = jnp.minimum(s4, bc4)
    w = ((k4 <= hi).astype(jnp.float32)
         - (lo > 0).astype(jnp.float32) * (k4 <= lo - 1).astype(jnp.float32))
    mass = jnp.sum(jnp.reshape(counts, (f, 1, 1, NBINS)) * w, axis=3)
    p = mass / float(n + c)
    pt_ref[...] = jnp.reshape(p * p, (f * NBINS, c))


def _hist(z, clt):
    n, f = z.shape
    c = clt.shape[1]
    return pl.pallas_call(
        _hist_body,
        out_shape=[
            jax.ShapeDtypeStruct((n, f), jnp.int32),
            jax.ShapeDtypeStruct((f * NBINS, c), jnp.float32),
        ],
    )(z, clt)


def _q_body(bz_ref, pt_ref, q_ref):
    bz = bz_ref[...]
    bn, f = bz.shape
    oh = (bz[:, :, None]
          == jax.lax.broadcasted_iota(jnp.int32, (bn, f, NBINS), 2))
    oh = jnp.reshape(oh.astype(jnp.float32), (bn, f * NBINS))
    dmass = jnp.dot(oh, pt_ref[...], preferred_element_type=jnp.float32)
    dm = jnp.sqrt(dmass)
    qq = 1.0 / (1.0 + dm)
    q_ref[...] = qq / jnp.sum(qq, axis=1, keepdims=True)


def _q(bz, pt, bn):
    n, f = bz.shape
    c = pt.shape[1]
    return pl.pallas_call(
        _q_body,
        grid=(n // bn,),
        in_specs=[
            pl.BlockSpec((bn, f), lambda i: (i, 0)),
            pl.BlockSpec((f * NBINS, c), lambda i: (0, 0)),
        ],
        out_specs=pl.BlockSpec((bn, c), lambda i: (i, 0)),
        out_shape=jax.ShapeDtypeStruct((n, c), jnp.float32),
    )(bz, pt)


def kernel(x, adj, M, W1, a_self1, a_neighs1, W2, a_self2, a_neighs2,
           cluster_layer):
    h1, ss1, sn1 = _proj(x, W1, a_self1, a_neighs1, 512)
    h1 = _att(ss1, jnp.reshape(sn1, (1, -1)), M, adj, h1, 256, False)
    h2, ss2, sn2 = _proj(h1, W2, a_self2, a_neighs2, 512)
    z = _att(ss2, jnp.reshape(sn2, (1, -1)), M, adj, h2, 256, True)
    a_pred = _apred(z, 256)
    bz, pt = _hist(z, jnp.transpose(cluster_layer))
    q = _q(bz, pt, 512)
    return (a_pred, z, q)
